# baseline (speedup 1.0000x reference)
import functools

import jax
import jax.numpy as jnp
from jax import lax
from jax.experimental import pallas as pl
from jax.experimental.pallas import tpu as pltpu
from jax.experimental.pallas import tpu_sc as plsc


def _fps_body(npoint, x_ref, y_ref, z_ref, cx_ref, cy_ref, cz_ref):
    X = x_ref[...]
    Y = y_ref[...]
    Z = z_ref[...]
    Bn, N = X.shape
    iota = jax.lax.broadcasted_iota(jnp.int32, (Bn, N), 1)
    lane = jax.lax.broadcasted_iota(jnp.int32, (Bn, 128), 1)

    def body(l, carry):
        dists, far, acx, acy, acz = carry
        onehot = iota == far
        cx = jnp.sum(jnp.where(onehot, X, 0.0), axis=1, keepdims=True)
        cy = jnp.sum(jnp.where(onehot, Y, 0.0), axis=1, keepdims=True)
        cz = jnp.sum(jnp.where(onehot, Z, 0.0), axis=1, keepdims=True)
        sel = lane == l
        acx = jnp.where(sel, cx, acx)
        acy = jnp.where(sel, cy, acy)
        acz = jnp.where(sel, cz, acz)
        dx = X - cx
        dy = Y - cy
        dz = Z - cz
        d = dx * dx + dy * dy + dz * dz
        dists = jnp.minimum(dists, d)
        m = jnp.max(dists, axis=1, keepdims=True)
        far = jnp.min(jnp.where(dists == m, iota, N), axis=1, keepdims=True)
        return dists, far, acx, acy, acz

    dists = jnp.full((Bn, N), 1e10, jnp.float32)
    far = jnp.zeros((Bn, 1), jnp.int32)
    zero128 = jnp.zeros((Bn, 128), jnp.float32)
    for g in range(npoint // 128):
        dists, far, acx, acy, acz = jax.lax.fori_loop(
            0, 128, body, (dists, far, zero128, zero128, zero128))
        cx_ref[:, g * 128:(g + 1) * 128] = acx
        cy_ref[:, g * 128:(g + 1) * 128] = acy
        cz_ref[:, g * 128:(g + 1) * 128] = acz


def _fps(X, Y, Z, npoint):
    Bn, N = X.shape
    out = jax.ShapeDtypeStruct((Bn, npoint), jnp.float32)
    return pl.pallas_call(
        functools.partial(_fps_body, npoint),
        out_shape=(out, out, out),
    )(X, Y, Z)


def _maskpack_body(r2, c_ref, p_ref, sel_ref, w_ref):
    C = c_ref[0]
    P = p_ref[0]
    Sel = sel_ref[...]
    Mc = C.shape[0]
    N = P.shape[1]

    aa = jnp.sum(C * C, axis=1, keepdims=True)
    bb = jnp.sum(P * P, axis=0, keepdims=True)
    ab = jnp.dot(C, P, preferred_element_type=jnp.float32,
                 precision=jax.lax.Precision.HIGHEST)
    d = aa + bb - 2.0 * ab
    iota = jax.lax.broadcasted_iota(jnp.int32, (Mc, N), 1)
    pw = jnp.left_shift(jnp.int32(1), iota % 16).astype(jnp.float32)
    bits = jnp.where(d <= r2, pw, 0.0)
    wlo = jnp.dot(bits, Sel[0], preferred_element_type=jnp.float32)
    whi = jnp.dot(bits, Sel[1], preferred_element_type=jnp.float32)
    w_ref[0] = wlo.astype(jnp.int32) | jnp.left_shift(whi.astype(jnp.int32), 16)


def _maskpack(Cc, Pts, Sel, r2, Mc=256):
    Bn, M, _ = Cc.shape
    N = Pts.shape[2]
    Wn = Sel.shape[2]
    grid = (Bn, M // Mc)
    return pl.pallas_call(
        functools.partial(_maskpack_body, r2),
        grid=grid,
        in_specs=[
            pl.BlockSpec((1, Mc, 8), lambda b, m: (b, m, 0)),
            pl.BlockSpec((1, 8, N), lambda b, m: (b, 0, 0)),
            pl.BlockSpec((2, N, Wn), lambda b, m: (0, 0, 0)),
        ],
        out_specs=pl.BlockSpec((1, Mc, Wn), lambda b, m: (b, m, 0)),
        out_shape=jax.ShapeDtypeStruct((Bn, M, Wn), jnp.int32),
    )(Cc, Pts, Sel)


def _sc_group(W, Tflat, N, K=32):
    BnM, Wn = W.shape
    BnN, Cw = Tflat.shape
    Bn = BnN // N
    NW = 32
    rpw = BnM // NW
    rows_per_batch = BnM // Bn
    CH = 32
    nsteps = Wn // 16
    mesh = plsc.VectorSubcoreMesh(core_axis_name="c", subcore_axis_name="s")

    @functools.partial(
        pl.kernel, mesh=mesh,
        compiler_params=pltpu.CompilerParams(use_tc_tiling_on_sc=False),
        out_type=pltpu.HBM((BnM, K, Cw), jnp.float32),
        scratch_types=[
            pltpu.VMEM((rpw, Wn), jnp.int32),
            pltpu.VMEM((CH, K, Cw), jnp.float32),
            pltpu.VMEM((160,), jnp.int32),
            pltpu.VMEM((160,), jnp.int32),
            pltpu.VMEM((64,), jnp.int32),
            pltpu.VMEM((32,), jnp.int32),
            pltpu.VMEM((16,), jnp.int32),
            pltpu.VMEM((16,), jnp.int32),
            pltpu.SemaphoreType.DMA,
        ],
    )
    def body(w_hbm, t_hbm, g_hbm, wv, ov, wposv, wvalv, idxv, idx32v,
             ntv, stv, sem):
        wid = lax.axis_index("c") * 16 + lax.axis_index("s")
        row0 = wid * rpw
        boff = (row0 // rows_per_batch) * N
        pltpu.sync_copy(w_hbm.at[pl.ds(row0, rpw)], wv)
        iota = lax.iota(jnp.int32, 16)

        def chunk_body(ch, _):
            def row_body(r, __):
                rl = ch * CH + r
                idxv[0:16] = jnp.zeros((16,), jnp.int32)
                ntv[0:16] = jnp.zeros((16,), jnp.int32)
                stv[0:16] = jnp.zeros((16,), jnp.int32)

                for s in range(nsteps):
                    wvv = wv[rl, s * 16:(s + 1) * 16]
                    for l in range(16):
                        w0 = wvv[l]

                        @pl.when(w0 != 0)
                        def _(w0=w0, s=s, l=l):
                            t0 = ntv[0:16][0]
                            wvalv[pl.ds(t0, 16)] = jnp.full(
                                (16,), w0, jnp.int32)
                            wposv[pl.ds(t0, 16)] = jnp.full(
                                (16,), s * 16 + l, jnp.int32)
                            ntv[0:16] = jnp.full((16,), t0 + 1, jnp.int32)

                nw = ntv[0:16][0]

                for t in range(K):
                    st = stv[0:16]
                    cnt = st[0]
                    ti = st[1]
                    wcur = st[2]
                    p = st[3]
                    active = ((wcur != 0) | (ti < nw)) & (cnt < K)

                    @pl.when(active)
                    def _():
                        ld = wvalv[pl.ds(ti, 16)]
                        lp = wposv[pl.ds(ti, 16)]
                        fetch = wcur == 0
                        w2 = jnp.where(fetch, ld[0], wcur)
                        p2 = jnp.where(fetch, lp[0], p)
                        ti2 = jnp.where(fetch, ti + 1, ti)
                        bb = w2 & (-w2)
                        bf = bb.astype(jnp.float32)
                        e = (lax.bitcast_convert_type(bf, jnp.int32)
                             >> 23) - 127
                        bp = jnp.where(bb < 0, jnp.int32(31), e)
                        idxv[pl.ds(cnt, 16)] = jnp.full(
                            (16,), p2 * 32 + bp, jnp.int32)
                        w3 = w2 & (w2 - 1)
                        stv[0:16] = jnp.where(
                            iota == 0, cnt + 1,
                            jnp.where(iota == 1, ti2,
                                      jnp.where(iota == 2, w3, p2)))

                cnt = stv[0:16][0]
                i0 = idxv[0:16][0]
                for h in range(K // 16):
                    sl = idxv[h * 16:(h + 1) * 16]
                    slot = h * 16 + iota
                    idx32v[h * 16:(h + 1) * 16] = jnp.where(
                        slot < cnt, sl, i0) + boff
                pltpu.async_copy(t_hbm.at[idx32v], ov.at[r], sem).wait()
                return 0

            lax.fori_loop(0, CH, row_body, 0, unroll=False)
            pltpu.sync_copy(ov, g_hbm.at[pl.ds(row0 + ch * CH, CH)])
            return 0

        lax.fori_loop(0, rpw // CH, chunk_body, 0, unroll=False)

    return body(W, Tflat)


def _group_body(r2, K, c_ref, cpad_ref, p_ref, t_ref, g_ref):
    C = c_ref[0]
    P = p_ref[0]
    T = t_ref[0]
    Cpad = cpad_ref[0]
    Mc = C.shape[0]
    N = P.shape[1]

    aa = jnp.sum(C * C, axis=1, keepdims=True)
    bb = jnp.sum(P * P, axis=0, keepdims=True)
    ab = jnp.dot(C, P, preferred_element_type=jnp.float32, precision=jax.lax.Precision.HIGHEST)
    d = aa + bb - 2.0 * ab
    mask = d <= r2

    iota = jax.lax.broadcasted_iota(jnp.int32, (Mc, N), 1)
    g0 = None
    for k in range(K):
        jk = jnp.min(jnp.where(mask, iota, N), axis=1, keepdims=True)
        onehot = (iota == jk).astype(jnp.float32)
        g = jnp.dot(onehot, T, preferred_element_type=jnp.float32, precision=jax.lax.Precision.HIGHEST) - Cpad
        if g0 is None:
            g0 = g
        else:
            g = jnp.where(jk < N, g, g0)
        g_ref[0, :, k, :] = g
        mask = jnp.logical_and(mask, iota != jk)


def _group(Cc, Cpad, Pts, Tbl, r2, K=32, Mc=256):
    Bn, M, _ = Cc.shape
    N = Pts.shape[2]
    Cw = Tbl.shape[2]
    grid = (Bn, M // Mc)
    return pl.pallas_call(
        functools.partial(_group_body, r2, K),
        grid=grid,
        in_specs=[
            pl.BlockSpec((1, Mc, 8), lambda b, m: (b, m, 0)),
            pl.BlockSpec((1, Mc, Cw), lambda b, m: (b, m, 0)),
            pl.BlockSpec((1, 8, N), lambda b, m: (b, 0, 0)),
            pl.BlockSpec((1, N, Cw), lambda b, m: (b, 0, 0)),
        ],
        out_specs=pl.BlockSpec((1, Mc, K, Cw), lambda b, m: (b, m, 0, 0)),
        out_shape=jax.ShapeDtypeStruct((Bn, M, K, Cw), jnp.float32),
    )(Cc, Cpad, Pts, Tbl)


def _mlp1_body(wa_ref, ba_ref, wb_ref, bb_ref, g_ref, cpe_ref, f_ref):
    Mc, K, Cw = g_ref.shape[1:]
    h = (g_ref[0] - cpe_ref[0]).reshape(Mc * K, Cw)
    h = jnp.maximum(jnp.dot(h, wa_ref[...], preferred_element_type=jnp.float32, precision=jax.lax.Precision.HIGHEST)
                    + ba_ref[...], 0.0)
    h = jnp.maximum(jnp.dot(h, wb_ref[...], preferred_element_type=jnp.float32, precision=jax.lax.Precision.HIGHEST)
                    + bb_ref[...], 0.0)
    f_ref[0] = jnp.max(h.reshape(Mc, K, h.shape[1]), axis=1)


def _mlp1(G, CpE, wa, ba, wb, bb, Mc=256):
    Bn, M, K, Cw = G.shape
    Co = wb.shape[1]
    grid = (Bn, M // Mc)
    return pl.pallas_call(
        _mlp1_body,
        grid=grid,
        in_specs=[
            pl.BlockSpec((wa.shape[0], Co), lambda b, m: (0, 0)),
            pl.BlockSpec((1, Co), lambda b, m: (0, 0)),
            pl.BlockSpec((Co, Co), lambda b, m: (0, 0)),
            pl.BlockSpec((1, Co), lambda b, m: (0, 0)),
            pl.BlockSpec((1, Mc, K, Cw), lambda b, m: (b, m, 0, 0)),
            pl.BlockSpec((1, Mc, K, Cw), lambda b, m: (b, m, 0, 0)),
        ],
        out_specs=pl.BlockSpec((1, Mc, Co), lambda b, m: (b, m, 0)),
        out_shape=jax.ShapeDtypeStruct((Bn, M, Co), jnp.float32),
    )(wa, ba.reshape(1, -1), wb, bb.reshape(1, -1), G, CpE)


def _mlp2_final_body(wa_ref, ba_ref, wb_ref, bb_ref, wz_ref, bz_ref,
                     g_ref, cpe_ref, o_ref):
    Mc, K, Cw = g_ref.shape[1:]
    h = (g_ref[0] - cpe_ref[0]).reshape(Mc * K, Cw)
    h = jnp.maximum(jnp.dot(h, wa_ref[...], preferred_element_type=jnp.float32, precision=jax.lax.Precision.HIGHEST)
                    + ba_ref[...], 0.0)
    h = jnp.maximum(jnp.dot(h, wb_ref[...], preferred_element_type=jnp.float32, precision=jax.lax.Precision.HIGHEST)
                    + bb_ref[...], 0.0)
    f2 = jnp.max(h.reshape(Mc, K, h.shape[1]), axis=1)
    feat = jnp.max(f2, axis=0, keepdims=True)
    o_ref[0] = jnp.dot(feat, wz_ref[...],
                       preferred_element_type=jnp.float32, precision=jax.lax.Precision.HIGHEST) + bz_ref[...]


def _mlp2_final(G, CpE, wa, ba, wb, bb, wz, bz):
    Bn, M, K, Cw = G.shape
    Ch = wb.shape[1]
    Z = wz.shape[1]
    return pl.pallas_call(
        _mlp2_final_body,
        grid=(Bn,),
        in_specs=[
            pl.BlockSpec((wa.shape[0], wa.shape[1]), lambda b: (0, 0)),
            pl.BlockSpec((1, wa.shape[1]), lambda b: (0, 0)),
            pl.BlockSpec((wb.shape[0], Ch), lambda b: (0, 0)),
            pl.BlockSpec((1, Ch), lambda b: (0, 0)),
            pl.BlockSpec((Ch, Z), lambda b: (0, 0)),
            pl.BlockSpec((1, Z), lambda b: (0, 0)),
            pl.BlockSpec((1, M, K, Cw), lambda b: (b, 0, 0, 0)),
            pl.BlockSpec((1, M, K, Cw), lambda b: (b, 0, 0, 0)),
        ],
        out_specs=pl.BlockSpec((1, 1, Z), lambda b: (b, 0, 0)),
        out_shape=jax.ShapeDtypeStruct((Bn, 1, Z), jnp.float32),
    )(wa, ba.reshape(1, -1), wb, bb.reshape(1, -1), wz, bz.reshape(1, -1), G,
      CpE).reshape(Bn, Z)


def kernel(x, w1a, b1a, w1b, b1b, w2a, b2a, w2b, b2b, wz, bz):
    Bn, N, _ = x.shape
    xt = jnp.transpose(x, (0, 2, 1))
    X, Y, Z = xt[:, 0], xt[:, 1], xt[:, 2]
    zcol = jnp.zeros((Bn, N), jnp.float32)

    CX1, CY1, CZ1 = _fps(X, Y, Z, 1024)
    P1 = jnp.stack([X, Y, Z, zcol, zcol, zcol, zcol, zcol], axis=1)
    T1 = jnp.stack([X, Y, Z, zcol, X, Y, Z, zcol] + [zcol] * 8,
                   axis=2)
    zc1 = jnp.zeros((Bn, 1024), jnp.float32)
    C1 = jnp.stack([CX1, CY1, CZ1, zc1, zc1, zc1, zc1, zc1], axis=2)
    g16 = jnp.arange(N) // 16
    gw1 = jnp.arange(N // 32)
    Sel1 = jnp.stack(
        [(g16[:, None] == 2 * gw1[None, :]).astype(jnp.float32),
         (g16[:, None] == 2 * gw1[None, :] + 1).astype(jnp.float32)])
    W1 = _maskpack(C1, P1, Sel1, r2=0.1 * 0.1)
    G1 = _sc_group(W1.reshape(Bn * 1024, N // 32),
                   T1.reshape(Bn * N, 16), N).reshape(Bn, 1024, 32, 16)
    C1w = jnp.concatenate(
        [C1, jnp.zeros((Bn, 1024, 8), jnp.float32)], axis=2)
    CpE1 = jnp.broadcast_to(C1w[:, :, None, :], (Bn, 1024, 32, 16))
    w1a_pad = jnp.concatenate(
        [w1a[0:3], jnp.zeros((1, 32), jnp.float32),
         w1a[3:6], jnp.zeros((9, 32), jnp.float32)], axis=0)
    f1 = _mlp1(G1, CpE1, w1a_pad, b1a, w1b, b1b)

    CX2, CY2, CZ2 = _fps(CX1, CY1, CZ1, 256)
    zc1b = jnp.zeros((Bn, 1024), jnp.float32)
    P2 = jnp.stack([CX1, CY1, CZ1, zc1b, zc1b, zc1b, zc1b, zc1b], axis=1)
    xyz1 = jnp.stack([CX1, CY1, CZ1], axis=2)
    T2 = jnp.concatenate(
        [xyz1, f1, jnp.zeros((Bn, 1024, 13), jnp.float32)], axis=2)
    zc2 = jnp.zeros((Bn, 256), jnp.float32)
    C2 = jnp.stack([CX2, CY2, CZ2, zc2, zc2, zc2, zc2, zc2], axis=2)
    g16b = jnp.arange(1024) // 16
    gw2 = jnp.arange(32)
    Sel2 = jnp.stack(
        [(g16b[:, None] == 2 * gw2[None, :]).astype(jnp.float32),
         (g16b[:, None] == 2 * gw2[None, :] + 1).astype(jnp.float32)])
    W2 = _maskpack(C2, P2, Sel2, r2=0.2 * 0.2)
    G2 = _sc_group(W2.reshape(Bn * 256, 32),
                   T2.reshape(Bn * 1024, 48), 1024).reshape(Bn, 256, 32, 48)
    Cp2 = jnp.concatenate(
        [jnp.stack([CX2, CY2, CZ2], axis=2),
         jnp.zeros((Bn, 256, 45), jnp.float32)], axis=2)
    CpE2 = jnp.broadcast_to(Cp2[:, :, None, :], (Bn, 256, 32, 48))
    w2a_pad = jnp.concatenate(
        [w2a, jnp.zeros((13, 32), jnp.float32)], axis=0)
    return _mlp2_final(G2, CpE2, w2a_pad, b2a, w2b, b2b, wz, bz)


# speedup vs baseline: 1.1568x; 1.1568x over previous
import functools

import jax
import jax.numpy as jnp
from jax import lax
from jax.experimental import pallas as pl
from jax.experimental.pallas import tpu as pltpu
from jax.experimental.pallas import tpu_sc as plsc


def _fps_body(npoint, x_ref, y_ref, z_ref, cx_ref, cy_ref, cz_ref):
    X = x_ref[...]
    Y = y_ref[...]
    Z = z_ref[...]
    Bn, N = X.shape
    iota = jax.lax.broadcasted_iota(jnp.int32, (Bn, N), 1)
    lane = jax.lax.broadcasted_iota(jnp.int32, (Bn, 128), 1)

    def body(l, carry):
        dists, far, acx, acy, acz = carry
        onehot = iota == far
        cx = jnp.sum(jnp.where(onehot, X, 0.0), axis=1, keepdims=True)
        cy = jnp.sum(jnp.where(onehot, Y, 0.0), axis=1, keepdims=True)
        cz = jnp.sum(jnp.where(onehot, Z, 0.0), axis=1, keepdims=True)
        sel = lane == l
        acx = jnp.where(sel, cx, acx)
        acy = jnp.where(sel, cy, acy)
        acz = jnp.where(sel, cz, acz)
        dx = X - cx
        dy = Y - cy
        dz = Z - cz
        d = dx * dx + dy * dy + dz * dz
        dists = jnp.minimum(dists, d)
        m = jnp.max(dists, axis=1, keepdims=True)
        far = jnp.min(jnp.where(dists == m, iota, N), axis=1, keepdims=True)
        return dists, far, acx, acy, acz

    dists = jnp.full((Bn, N), 1e10, jnp.float32)
    far = jnp.zeros((Bn, 1), jnp.int32)
    zero128 = jnp.zeros((Bn, 128), jnp.float32)
    for g in range(npoint // 128):
        dists, far, acx, acy, acz = jax.lax.fori_loop(
            0, 128, body, (dists, far, zero128, zero128, zero128))
        cx_ref[:, g * 128:(g + 1) * 128] = acx
        cy_ref[:, g * 128:(g + 1) * 128] = acy
        cz_ref[:, g * 128:(g + 1) * 128] = acz


def _fps(X, Y, Z, npoint):
    Bn, N = X.shape
    out = jax.ShapeDtypeStruct((Bn, npoint), jnp.float32)
    return pl.pallas_call(
        functools.partial(_fps_body, npoint),
        out_shape=(out, out, out),
    )(X, Y, Z)


def _maskpack_body(r2, c_ref, p_ref, sel_ref, w_ref):
    C = c_ref[0]
    P = p_ref[0]
    Sel = sel_ref[...]
    Mc = C.shape[0]
    N = P.shape[1]

    aa = jnp.sum(C * C, axis=1, keepdims=True)
    bb = jnp.sum(P * P, axis=0, keepdims=True)
    ab = jnp.dot(C, P, preferred_element_type=jnp.float32,
                 precision=jax.lax.Precision.HIGHEST)
    d = aa + bb - 2.0 * ab
    iota = jax.lax.broadcasted_iota(jnp.int32, (Mc, N), 1)
    pw = jnp.left_shift(jnp.int32(1), iota % 16).astype(jnp.float32)
    bits = jnp.where(d <= r2, pw, 0.0)
    wlo = jnp.dot(bits, Sel[0], preferred_element_type=jnp.float32)
    whi = jnp.dot(bits, Sel[1], preferred_element_type=jnp.float32)
    w_ref[0] = wlo.astype(jnp.int32) | jnp.left_shift(whi.astype(jnp.int32), 16)


def _maskpack(Cc, Pts, Sel, r2, Mc=256):
    Bn, M, _ = Cc.shape
    N = Pts.shape[2]
    Wn = Sel.shape[2]
    grid = (Bn, M // Mc)
    return pl.pallas_call(
        functools.partial(_maskpack_body, r2),
        grid=grid,
        in_specs=[
            pl.BlockSpec((1, Mc, 8), lambda b, m: (b, m, 0)),
            pl.BlockSpec((1, 8, N), lambda b, m: (b, 0, 0)),
            pl.BlockSpec((2, N, Wn), lambda b, m: (0, 0, 0)),
        ],
        out_specs=pl.BlockSpec((1, Mc, Wn), lambda b, m: (b, m, 0)),
        out_shape=jax.ShapeDtypeStruct((Bn, M, Wn), jnp.int32),
    )(Cc, Pts, Sel)


def _sc_group(W, Tflat, N, K=32):
    BnM, Wn = W.shape
    BnN, Cw = Tflat.shape
    Bn = BnN // N
    NW = 32
    rpw = BnM // NW
    rows_per_batch = BnM // Bn
    CH = 32
    nsteps = Wn // 16
    mesh = plsc.VectorSubcoreMesh(core_axis_name="c", subcore_axis_name="s")

    @functools.partial(
        pl.kernel, mesh=mesh,
        compiler_params=pltpu.CompilerParams(use_tc_tiling_on_sc=False),
        out_type=pltpu.HBM((BnM, K, Cw), jnp.float32),
        scratch_types=[
            pltpu.VMEM((rpw, Wn), jnp.int32),
            pltpu.VMEM((CH, K, Cw), jnp.float32),
            pltpu.VMEM((160,), jnp.int32),
            pltpu.VMEM((160,), jnp.int32),
            pltpu.VMEM((64,), jnp.int32),
            pltpu.VMEM((CH, 32), jnp.int32),
            pltpu.VMEM((16,), jnp.int32),
            pltpu.VMEM((16,), jnp.int32),
            pltpu.SemaphoreType.DMA,
        ],
    )
    def body(w_hbm, t_hbm, g_hbm, wv, ov, wposv, wvalv, idxv, idx32v,
             ntv, stv, sem):
        wid = lax.axis_index("c") * 16 + lax.axis_index("s")
        row0 = wid * rpw
        boff = (row0 // rows_per_batch) * N
        pltpu.sync_copy(w_hbm.at[pl.ds(row0, rpw)], wv)
        iota = lax.iota(jnp.int32, 16)

        def chunk_body(ch, _):
            def row_body(r, __):
                rl = ch * CH + r
                idxv[0:16] = jnp.zeros((16,), jnp.int32)
                ntv[0:16] = jnp.zeros((16,), jnp.int32)
                stv[0:16] = jnp.zeros((16,), jnp.int32)

                for s in range(nsteps):
                    wvv = wv[rl, s * 16:(s + 1) * 16]
                    for l in range(16):
                        w0 = wvv[l]

                        @pl.when(w0 != 0)
                        def _(w0=w0, s=s, l=l):
                            t0 = ntv[0:16][0]
                            wvalv[pl.ds(t0, 16)] = jnp.full(
                                (16,), w0, jnp.int32)
                            wposv[pl.ds(t0, 16)] = jnp.full(
                                (16,), s * 16 + l, jnp.int32)
                            ntv[0:16] = jnp.full((16,), t0 + 1, jnp.int32)

                nw = ntv[0:16][0]

                for t in range(K):
                    st = stv[0:16]
                    cnt = st[0]
                    ti = st[1]
                    wcur = st[2]
                    p = st[3]
                    active = ((wcur != 0) | (ti < nw)) & (cnt < K)

                    @pl.when(active)
                    def _():
                        ld = wvalv[pl.ds(ti, 16)]
                        lp = wposv[pl.ds(ti, 16)]
                        fetch = wcur == 0
                        w2 = jnp.where(fetch, ld[0], wcur)
                        p2 = jnp.where(fetch, lp[0], p)
                        ti2 = jnp.where(fetch, ti + 1, ti)
                        bb = w2 & (-w2)
                        bf = bb.astype(jnp.float32)
                        e = (lax.bitcast_convert_type(bf, jnp.int32)
                             >> 23) - 127
                        bp = jnp.where(bb < 0, jnp.int32(31), e)
                        idxv[pl.ds(cnt, 16)] = jnp.full(
                            (16,), p2 * 32 + bp, jnp.int32)
                        w3 = w2 & (w2 - 1)
                        stv[0:16] = jnp.where(
                            iota == 0, cnt + 1,
                            jnp.where(iota == 1, ti2,
                                      jnp.where(iota == 2, w3, p2)))

                cnt = stv[0:16][0]
                i0 = idxv[0:16][0]
                for h in range(K // 16):
                    sl = idxv[h * 16:(h + 1) * 16]
                    slot = h * 16 + iota
                    idx32v[r, h * 16:(h + 1) * 16] = jnp.where(
                        slot < cnt, sl, i0) + boff
                pltpu.async_copy(t_hbm.at[idx32v.at[r]], ov.at[r], sem)
                return 0

            lax.fori_loop(0, CH, row_body, 0, unroll=False)
            pltpu.make_async_copy(
                g_hbm.at[pl.ds(row0 + ch * CH, CH)], ov, sem).wait()
            pltpu.sync_copy(ov, g_hbm.at[pl.ds(row0 + ch * CH, CH)])
            return 0

        lax.fori_loop(0, rpw // CH, chunk_body, 0, unroll=False)

    return body(W, Tflat)


def _group_body(r2, K, c_ref, cpad_ref, p_ref, t_ref, g_ref):
    C = c_ref[0]
    P = p_ref[0]
    T = t_ref[0]
    Cpad = cpad_ref[0]
    Mc = C.shape[0]
    N = P.shape[1]

    aa = jnp.sum(C * C, axis=1, keepdims=True)
    bb = jnp.sum(P * P, axis=0, keepdims=True)
    ab = jnp.dot(C, P, preferred_element_type=jnp.float32, precision=jax.lax.Precision.HIGHEST)
    d = aa + bb - 2.0 * ab
    mask = d <= r2

    iota = jax.lax.broadcasted_iota(jnp.int32, (Mc, N), 1)
    g0 = None
    for k in range(K):
        jk = jnp.min(jnp.where(mask, iota, N), axis=1, keepdims=True)
        onehot = (iota == jk).astype(jnp.float32)
        g = jnp.dot(onehot, T, preferred_element_type=jnp.float32, precision=jax.lax.Precision.HIGHEST) - Cpad
        if g0 is None:
            g0 = g
        else:
            g = jnp.where(jk < N, g, g0)
        g_ref[0, :, k, :] = g
        mask = jnp.logical_and(mask, iota != jk)


def _group(Cc, Cpad, Pts, Tbl, r2, K=32, Mc=256):
    Bn, M, _ = Cc.shape
    N = Pts.shape[2]
    Cw = Tbl.shape[2]
    grid = (Bn, M // Mc)
    return pl.pallas_call(
        functools.partial(_group_body, r2, K),
        grid=grid,
        in_specs=[
            pl.BlockSpec((1, Mc, 8), lambda b, m: (b, m, 0)),
            pl.BlockSpec((1, Mc, Cw), lambda b, m: (b, m, 0)),
            pl.BlockSpec((1, 8, N), lambda b, m: (b, 0, 0)),
            pl.BlockSpec((1, N, Cw), lambda b, m: (b, 0, 0)),
        ],
        out_specs=pl.BlockSpec((1, Mc, K, Cw), lambda b, m: (b, m, 0, 0)),
        out_shape=jax.ShapeDtypeStruct((Bn, M, K, Cw), jnp.float32),
    )(Cc, Cpad, Pts, Tbl)


def _mlp1_body(wa_ref, ba_ref, wb_ref, bb_ref, g_ref, cpe_ref, f_ref):
    Mc, K, Cw = g_ref.shape[1:]
    h = (g_ref[0] - cpe_ref[0]).reshape(Mc * K, Cw)
    h = jnp.maximum(jnp.dot(h, wa_ref[...], preferred_element_type=jnp.float32, precision=jax.lax.Precision.HIGHEST)
                    + ba_ref[...], 0.0)
    h = jnp.maximum(jnp.dot(h, wb_ref[...], preferred_element_type=jnp.float32, precision=jax.lax.Precision.HIGHEST)
                    + bb_ref[...], 0.0)
    f_ref[0] = jnp.max(h.reshape(Mc, K, h.shape[1]), axis=1)


def _mlp1(G, CpE, wa, ba, wb, bb, Mc=256):
    Bn, M, K, Cw = G.shape
    Co = wb.shape[1]
    grid = (Bn, M // Mc)
    return pl.pallas_call(
        _mlp1_body,
        grid=grid,
        in_specs=[
            pl.BlockSpec((wa.shape[0], Co), lambda b, m: (0, 0)),
            pl.BlockSpec((1, Co), lambda b, m: (0, 0)),
            pl.BlockSpec((Co, Co), lambda b, m: (0, 0)),
            pl.BlockSpec((1, Co), lambda b, m: (0, 0)),
            pl.BlockSpec((1, Mc, K, Cw), lambda b, m: (b, m, 0, 0)),
            pl.BlockSpec((1, Mc, K, Cw), lambda b, m: (b, m, 0, 0)),
        ],
        out_specs=pl.BlockSpec((1, Mc, Co), lambda b, m: (b, m, 0)),
        out_shape=jax.ShapeDtypeStruct((Bn, M, Co), jnp.float32),
    )(wa, ba.reshape(1, -1), wb, bb.reshape(1, -1), G, CpE)


def _mlp2_final_body(wa_ref, ba_ref, wb_ref, bb_ref, wz_ref, bz_ref,
                     g_ref, cpe_ref, o_ref):
    Mc, K, Cw = g_ref.shape[1:]
    h = (g_ref[0] - cpe_ref[0]).reshape(Mc * K, Cw)
    h = jnp.maximum(jnp.dot(h, wa_ref[...], preferred_element_type=jnp.float32, precision=jax.lax.Precision.HIGHEST)
                    + ba_ref[...], 0.0)
    h = jnp.maximum(jnp.dot(h, wb_ref[...], preferred_element_type=jnp.float32, precision=jax.lax.Precision.HIGHEST)
                    + bb_ref[...], 0.0)
    f2 = jnp.max(h.reshape(Mc, K, h.shape[1]), axis=1)
    feat = jnp.max(f2, axis=0, keepdims=True)
    o_ref[0] = jnp.dot(feat, wz_ref[...],
                       preferred_element_type=jnp.float32, precision=jax.lax.Precision.HIGHEST) + bz_ref[...]


def _mlp2_final(G, CpE, wa, ba, wb, bb, wz, bz):
    Bn, M, K, Cw = G.shape
    Ch = wb.shape[1]
    Z = wz.shape[1]
    return pl.pallas_call(
        _mlp2_final_body,
        grid=(Bn,),
        in_specs=[
            pl.BlockSpec((wa.shape[0], wa.shape[1]), lambda b: (0, 0)),
            pl.BlockSpec((1, wa.shape[1]), lambda b: (0, 0)),
            pl.BlockSpec((wb.shape[0], Ch), lambda b: (0, 0)),
            pl.BlockSpec((1, Ch), lambda b: (0, 0)),
            pl.BlockSpec((Ch, Z), lambda b: (0, 0)),
            pl.BlockSpec((1, Z), lambda b: (0, 0)),
            pl.BlockSpec((1, M, K, Cw), lambda b: (b, 0, 0, 0)),
            pl.BlockSpec((1, M, K, Cw), lambda b: (b, 0, 0, 0)),
        ],
        out_specs=pl.BlockSpec((1, 1, Z), lambda b: (b, 0, 0)),
        out_shape=jax.ShapeDtypeStruct((Bn, 1, Z), jnp.float32),
    )(wa, ba.reshape(1, -1), wb, bb.reshape(1, -1), wz, bz.reshape(1, -1), G,
      CpE).reshape(Bn, Z)


def kernel(x, w1a, b1a, w1b, b1b, w2a, b2a, w2b, b2b, wz, bz):
    Bn, N, _ = x.shape
    xt = jnp.transpose(x, (0, 2, 1))
    X, Y, Z = xt[:, 0], xt[:, 1], xt[:, 2]
    zcol = jnp.zeros((Bn, N), jnp.float32)

    CX1, CY1, CZ1 = _fps(X, Y, Z, 1024)
    P1 = jnp.stack([X, Y, Z, zcol, zcol, zcol, zcol, zcol], axis=1)
    T1 = jnp.stack([X, Y, Z, zcol, X, Y, Z, zcol] + [zcol] * 8,
                   axis=2)
    zc1 = jnp.zeros((Bn, 1024), jnp.float32)
    C1 = jnp.stack([CX1, CY1, CZ1, zc1, zc1, zc1, zc1, zc1], axis=2)
    g16 = jnp.arange(N) // 16
    gw1 = jnp.arange(N // 32)
    Sel1 = jnp.stack(
        [(g16[:, None] == 2 * gw1[None, :]).astype(jnp.float32),
         (g16[:, None] == 2 * gw1[None, :] + 1).astype(jnp.float32)])
    W1 = _maskpack(C1, P1, Sel1, r2=0.1 * 0.1)
    G1 = _sc_group(W1.reshape(Bn * 1024, N // 32),
                   T1.reshape(Bn * N, 16), N).reshape(Bn, 1024, 32, 16)
    C1w = jnp.concatenate(
        [C1, jnp.zeros((Bn, 1024, 8), jnp.float32)], axis=2)
    CpE1 = jnp.broadcast_to(C1w[:, :, None, :], (Bn, 1024, 32, 16))
    w1a_pad = jnp.concatenate(
        [w1a[0:3], jnp.zeros((1, 32), jnp.float32),
         w1a[3:6], jnp.zeros((9, 32), jnp.float32)], axis=0)
    f1 = _mlp1(G1, CpE1, w1a_pad, b1a, w1b, b1b)

    CX2, CY2, CZ2 = _fps(CX1, CY1, CZ1, 256)
    zc1b = jnp.zeros((Bn, 1024), jnp.float32)
    P2 = jnp.stack([CX1, CY1, CZ1, zc1b, zc1b, zc1b, zc1b, zc1b], axis=1)
    xyz1 = jnp.stack([CX1, CY1, CZ1], axis=2)
    T2 = jnp.concatenate(
        [xyz1, f1, jnp.zeros((Bn, 1024, 13), jnp.float32)], axis=2)
    zc2 = jnp.zeros((Bn, 256), jnp.float32)
    C2 = jnp.stack([CX2, CY2, CZ2, zc2, zc2, zc2, zc2, zc2], axis=2)
    g16b = jnp.arange(1024) // 16
    gw2 = jnp.arange(32)
    Sel2 = jnp.stack(
        [(g16b[:, None] == 2 * gw2[None, :]).astype(jnp.float32),
         (g16b[:, None] == 2 * gw2[None, :] + 1).astype(jnp.float32)])
    W2 = _maskpack(C2, P2, Sel2, r2=0.2 * 0.2)
    G2 = _sc_group(W2.reshape(Bn * 256, 32),
                   T2.reshape(Bn * 1024, 48), 1024).reshape(Bn, 256, 32, 48)
    Cp2 = jnp.concatenate(
        [jnp.stack([CX2, CY2, CZ2], axis=2),
         jnp.zeros((Bn, 256, 45), jnp.float32)], axis=2)
    CpE2 = jnp.broadcast_to(Cp2[:, :, None, :], (Bn, 256, 32, 48))
    w2a_pad = jnp.concatenate(
        [w2a, jnp.zeros((13, 32), jnp.float32)], axis=0)
    return _mlp2_final(G2, CpE2, w2a_pad, b2a, w2b, b2b, wz, bz)
